# Initial kernel scaffold; baseline (speedup 1.0000x reference)
import jax
import jax.numpy as jnp
from jax.experimental import pallas as pl


def kernel(X, edge_index, batch, W1, b1, W2, b2, Wm1, bm1, Wm2, bm2):
    raise NotImplementedError("write your pallas kernel here")


# speedup vs baseline: 8.6838x; 8.6838x over previous
import functools

import jax
import jax.numpy as jnp
from jax import lax
from jax.experimental import pallas as pl
from jax.experimental.pallas import tpu as pltpu
from jax.experimental.pallas import tpu_sc as plsc

N = 10000
E = 320000
D = 128
G = 64
H_MLP = 64

NPAD = 10112
CH = 128
NCH = 160
EP = 16 * NCH * CH
CPT = NCH // 2
R = 400
GRID = N // R

_mesh = plsc.VectorSubcoreMesh(core_axis_name="c", subcore_axis_name="s")


def _deg_body(dst_hbm, zero1_hbm, out_hbm, idx_v, deg_v):
    c = lax.axis_index("c")
    s = lax.axis_index("s")
    wid = s * 2 + c
    pltpu.sync_copy(dst_hbm.at[pl.ds(wid * (CPT * CH), CPT * CH)], idx_v)
    pltpu.sync_copy(zero1_hbm, deg_v)
    ones16 = jnp.ones((16,), jnp.float32)

    def body(i, carry):
        idx16 = idx_v[pl.ds(i * 16, 16)]
        plsc.addupdate_scatter(deg_v, [idx16], ones16)
        return carry

    lax.fori_loop(0, CPT * CH // 16, body, 0)
    pltpu.sync_copy(deg_v, out_hbm.at[wid])


_deg_kernel = pl.kernel(
    _deg_body,
    out_type=jax.ShapeDtypeStruct((32, NPAD), jnp.float32),
    mesh=_mesh,
    scratch_types=[
        pltpu.VMEM((CPT * CH,), jnp.int32),
        pltpu.VMEM((NPAD,), jnp.float32),
    ],
    compiler_params=pltpu.CompilerParams(needs_layout_passes=False),
)


def _agg_body(scaled_hbm, srcp_hbm, dstp_hbm, zero2_hbm, out_hbm,
              srcv, dstv, rows_v, acc_spmem, gsem):
    c = lax.axis_index("c")
    s = lax.axis_index("s")
    pltpu.sync_copy(srcp_hbm.at[s, pl.ds(c * CPT, CPT)], srcv)
    pltpu.sync_copy(dstp_hbm.at[s, pl.ds(c * CPT, CPT)], dstv)
    pltpu.sync_copy(zero2_hbm, acc_spmem.at[pl.ds(s * (NPAD // 16), NPAD // 16)])
    plsc.subcore_barrier()

    def body(j, carry):
        pltpu.async_copy(scaled_hbm.at[srcv.at[j]], rows_v, gsem).wait()
        pltpu.sync_copy(rows_v, acc_spmem.at[dstv.at[j]], add=True)
        return carry

    lax.fori_loop(0, CPT, body, 0)
    plsc.subcore_barrier()
    pltpu.sync_copy(acc_spmem.at[pl.ds(s * (NPAD // 16), NPAD // 16)],
                    out_hbm.at[c, pl.ds(s * (NPAD // 16), NPAD // 16)])


_agg_kernel = pl.kernel(
    _agg_body,
    out_type=jax.ShapeDtypeStruct((2, NPAD, D), jnp.float32),
    mesh=_mesh,
    scratch_types=[
        pltpu.VMEM((CPT, CH), jnp.int32),
        pltpu.VMEM((CPT, CH), jnp.int32),
        pltpu.VMEM((CH, D), jnp.float32),
        pltpu.VMEM_SHARED((NPAD, D), jnp.float32),
        pltpu.SemaphoreType.DMA,
    ],
    compiler_params=pltpu.CompilerParams(needs_layout_passes=False),
)


def _scale1_body(x_ref, w_ref, degp_ref, scaled_ref, dis_ref):
    h = jnp.dot(x_ref[...], w_ref[...], preferred_element_type=jnp.float32)
    degsum = jnp.sum(degp_ref[...], axis=1, keepdims=True)
    dis = lax.rsqrt(degsum + 1.0)
    scaled_ref[...] = h * dis
    dis_ref[...] = dis


def _layer2_body(a0_ref, a1_ref, s1_ref, dis_ref, b1_ref, w2_ref, scaled2_ref):
    pre = a0_ref[0] + a1_ref[0] + s1_ref[...]
    out1 = jnp.maximum(dis_ref[...] * pre + b1_ref[...], 0.0)
    h2 = jnp.dot(out1, w2_ref[...], preferred_element_type=jnp.float32)
    scaled2_ref[...] = h2 * dis_ref[...]


def _final_body(a0_ref, a1_ref, s2_ref, dis_ref, b2_ref, batch_ref,
                wm1_ref, bm1_ref, wm2_ref, bm2_ref, o_ref, psum_scr, cnt_scr):
    i = pl.program_id(0)

    @pl.when(i == 0)
    def _():
        psum_scr[...] = jnp.zeros((G, D), jnp.float32)
        cnt_scr[...] = jnp.zeros((G, 1), jnp.float32)

    pre = a0_ref[0] + a1_ref[0] + s2_ref[...]
    out2 = dis_ref[...] * pre + b2_ref[...]
    gid = lax.broadcasted_iota(jnp.int32, (R, G), 1)
    onehot = (gid == batch_ref[...]).astype(jnp.float32)
    dnums = (((0,), (0,)), ((), ()))
    psum_scr[...] += lax.dot_general(onehot, out2, dnums,
                                     preferred_element_type=jnp.float32)
    cnt_scr[...] += lax.dot_general(onehot, jnp.ones((R, 1), jnp.float32),
                                    dnums,
                                    preferred_element_type=jnp.float32)

    @pl.when(i == GRID - 1)
    def _():
        pooled = psum_scr[...] / jnp.maximum(cnt_scr[...], 1.0)
        m = jnp.maximum(
            jnp.dot(pooled, wm1_ref[...],
                    preferred_element_type=jnp.float32) + bm1_ref[...], 0.0)
        m = jnp.dot(m, wm2_ref[...],
                    preferred_element_type=jnp.float32) + bm2_ref[...]
        o_ref[...] = jax.nn.sigmoid(m)


def kernel(X, edge_index, batch, W1, b1, W2, b2, Wm1, bm1, Wm2, bm2):
    f32 = jnp.float32
    src = edge_index[0]
    dst = edge_index[1]
    pad = EP - E
    src_p = jnp.pad(src, (0, pad)).reshape(16, NCH, CH)
    dst_p = jnp.pad(dst, (0, pad), constant_values=N).reshape(16, NCH, CH)
    dst_flat = dst_p.reshape(EP)
    zero1 = jnp.zeros((NPAD,), f32)
    zero2 = jnp.zeros((NPAD // 16, D), f32)

    degp = _deg_kernel(dst_flat, zero1).T[:N]

    scaled1, dis = pl.pallas_call(
        _scale1_body,
        grid=(GRID,),
        in_specs=[
            pl.BlockSpec((R, D), lambda i: (i, 0)),
            pl.BlockSpec((D, D), lambda i: (0, 0)),
            pl.BlockSpec((R, 32), lambda i: (i, 0)),
        ],
        out_specs=[
            pl.BlockSpec((R, D), lambda i: (i, 0)),
            pl.BlockSpec((R, 1), lambda i: (i, 0)),
        ],
        out_shape=[
            jax.ShapeDtypeStruct((N, D), f32),
            jax.ShapeDtypeStruct((N, 1), f32),
        ],
    )(X, W1, degp)

    agg1 = _agg_kernel(scaled1, src_p, dst_p, zero2)[:, :N]

    scaled2 = pl.pallas_call(
        _layer2_body,
        grid=(GRID,),
        in_specs=[
            pl.BlockSpec((1, R, D), lambda i: (0, i, 0)),
            pl.BlockSpec((1, R, D), lambda i: (1, i, 0)),
            pl.BlockSpec((R, D), lambda i: (i, 0)),
            pl.BlockSpec((R, 1), lambda i: (i, 0)),
            pl.BlockSpec((1, D), lambda i: (0, 0)),
            pl.BlockSpec((D, D), lambda i: (0, 0)),
        ],
        out_specs=pl.BlockSpec((R, D), lambda i: (i, 0)),
        out_shape=jax.ShapeDtypeStruct((N, D), f32),
    )(agg1, agg1, scaled1, dis, b1.reshape(1, D), W2)

    agg2 = _agg_kernel(scaled2, src_p, dst_p, zero2)[:, :N]

    out = pl.pallas_call(
        _final_body,
        grid=(GRID,),
        in_specs=[
            pl.BlockSpec((1, R, D), lambda i: (0, i, 0)),
            pl.BlockSpec((1, R, D), lambda i: (1, i, 0)),
            pl.BlockSpec((R, D), lambda i: (i, 0)),
            pl.BlockSpec((R, 1), lambda i: (i, 0)),
            pl.BlockSpec((1, D), lambda i: (0, 0)),
            pl.BlockSpec((R, 1), lambda i: (i, 0)),
            pl.BlockSpec((D, H_MLP), lambda i: (0, 0)),
            pl.BlockSpec((1, H_MLP), lambda i: (0, 0)),
            pl.BlockSpec((H_MLP, 1), lambda i: (0, 0)),
            pl.BlockSpec((1, 1), lambda i: (0, 0)),
        ],
        out_specs=pl.BlockSpec((G, 1), lambda i: (0, 0)),
        out_shape=jax.ShapeDtypeStruct((G, 1), f32),
        scratch_shapes=[
            pltpu.VMEM((G, D), f32),
            pltpu.VMEM((G, 1), f32),
        ],
        compiler_params=pltpu.CompilerParams(
            dimension_semantics=("arbitrary",)),
    )(agg2, agg2, scaled2, dis, b2.reshape(1, D), batch.reshape(N, 1),
      Wm1, bm1.reshape(1, H_MLP), Wm2, bm2.reshape(1, 1))

    return out
